# Initial kernel scaffold; baseline (speedup 1.0000x reference)
import jax
import jax.numpy as jnp
from jax.experimental import pallas as pl


def kernel(attention_weight):
    raise NotImplementedError("write your pallas kernel here")


# speedup vs baseline: 1.4301x; 1.4301x over previous
import jax
import jax.numpy as jnp
from jax.experimental import pallas as pl

_K = 8
_H = 16
_T = 2048
_S = 2048
_SAMPLED_HEAD = 12
_SLAB = 8


def _update_kernel(slab_ref, out_ref):
    slab = slab_ref[...]
    x = slab[:, _SLAB - 1, :]

    m = jnp.max(x, axis=-1, keepdims=True)
    idx = jax.lax.broadcasted_iota(jnp.int32, x.shape, 1)
    arg = jnp.min(jnp.where(x == m, idx, _S), axis=-1)

    eq = (arg[:, None] == arg[None, :]).astype(jnp.int32)
    maxcount = jnp.max(jnp.sum(eq, axis=1))
    cond = maxcount <= _K

    row = x[_SAMPLED_HEAD, :]
    newx = jnp.where(cond, jnp.broadcast_to(row[None, :], x.shape), x)

    t_idx = jax.lax.broadcasted_iota(jnp.int32, slab.shape, 1)
    out_ref[...] = jnp.where(t_idx == _SLAB - 1, newx[:, None, :], slab)


def kernel(attention_weight):
    aw = attention_weight.reshape(_H, _T, _S)
    blk = (_H, _SLAB, _S)
    last_blk = (_T - _SLAB) // _SLAB
    out = pl.pallas_call(
        _update_kernel,
        grid=(1,),
        in_specs=[pl.BlockSpec(blk, lambda i: (0, last_blk, 0))],
        out_specs=pl.BlockSpec(blk, lambda i: (0, last_blk, 0)),
        out_shape=jax.ShapeDtypeStruct((_H, _T, _S), jnp.float32),
        input_output_aliases={0: 0},
    )(aw)
    return out.reshape(1, _H, _T, _S)
